# Initial kernel scaffold; baseline (speedup 1.0000x reference)
import jax
import jax.numpy as jnp
from jax.experimental import pallas as pl


def kernel(transformer_logits, W, b):
    raise NotImplementedError("write your pallas kernel here")


# speedup vs baseline: 1.0821x; 1.0821x over previous
import jax
import jax.numpy as jnp
from jax.experimental import pallas as pl
from jax.experimental.pallas import tpu as pltpu

_BINS = 4096
_ADIM = 8
_OUT_DIM = _BINS * (_ADIM + 1)
_BN = 1024


def _matmul_body(x_ref, w_ref, b_ref, o_ref):
    o_ref[...] = (
        jnp.dot(x_ref[...], w_ref[...], preferred_element_type=jnp.float32)
        + b_ref[...]
    )


def _sample_gather_body(bins_ref, gmb_ref, resid_ref, sel_ref, selres_ref):
    z = bins_ref[...] + gmb_ref[...]
    sel = jnp.argmax(z, axis=-1).astype(jnp.int32)
    sel_ref[...] = sel[:, None]
    bs = bins_ref.shape[0]
    cols = jax.lax.broadcasted_iota(jnp.int32, (bs, _BINS * _ADIM), 1)
    resid = resid_ref[...]
    parts = []
    for c in range(_ADIM):
        m = cols == sel[:, None] * _ADIM + c
        parts.append(jnp.sum(jnp.where(m, resid, 0.0), axis=1, keepdims=True))
    selres_ref[...] = jnp.concatenate(parts, axis=1)


def kernel(transformer_logits, W, b):
    batch, seq, num_bins = transformer_logits.shape
    bs = batch * seq
    x2d = transformer_logits.reshape(bs, num_bins)
    b2d = b.reshape(1, _OUT_DIM)

    xfull = pl.pallas_call(
        _matmul_body,
        grid=(_OUT_DIM // _BN,),
        in_specs=[
            pl.BlockSpec((bs, num_bins), lambda j: (0, 0)),
            pl.BlockSpec((num_bins, _BN), lambda j: (0, j)),
            pl.BlockSpec((1, _BN), lambda j: (0, j)),
        ],
        out_specs=pl.BlockSpec((bs, _BN), lambda j: (0, j)),
        out_shape=jax.ShapeDtypeStruct((bs, _OUT_DIM), jnp.float32),
        compiler_params=pltpu.CompilerParams(
            dimension_semantics=("parallel",)
        ),
    )(x2d, W, b2d)

    bins_logits = xfull[:, :num_bins]
    resid = xfull[:, num_bins:]
    gumbel = jax.random.gumbel(jax.random.key(42), (bs, num_bins), jnp.float32)

    sel, selres = pl.pallas_call(
        _sample_gather_body,
        out_shape=(
            jax.ShapeDtypeStruct((bs, 1), jnp.int32),
            jax.ShapeDtypeStruct((bs, _ADIM), jnp.float32),
        ),
    )(bins_logits, gumbel, resid)

    return (
        sel.reshape(batch, seq, 1),
        selres.reshape(batch, seq, _ADIM),
        resid.reshape(batch, seq, num_bins, _ADIM),
        bins_logits.reshape(batch, seq, num_bins),
    )
